# baseline (speedup 1.0000x reference)
import functools

import jax
import jax.numpy as jnp
from jax import lax
from jax.experimental import pallas as pl
from jax.experimental.pallas import tpu as pltpu
from jax.experimental.pallas import tpu_sc as plsc

VOCAB = 100000
HIDDEN = 768
MAX_POS = 2048
BATCH = 4
SEQ = 2048
TOK = BATCH * SEQ

NC, NS = 2, 16
NW = NC * NS
TPW = TOK // NW
C = 32
NCHUNK = TPW // C
GROUPS = HIDDEN // 16


def _embed_body(ids_hbm, tt_hbm, word_hbm, type_hbm, pos_hbm, out_hbm,
                widx_v, tidx_v, acc_v, pos_v, typ_v, sem, sem2, sem3):
    wid = lax.axis_index("s") * NC + lax.axis_index("c")
    base0 = wid * TPW
    for j in range(NCHUNK):
        base = base0 + j * C
        s_off = base % SEQ
        pltpu.sync_copy(ids_hbm.at[pl.ds(base, C)], widx_v)
        pltpu.sync_copy(tt_hbm.at[pl.ds(base, C)], tidx_v)
        cp_pos = pltpu.async_copy(pos_hbm.at[pl.ds(s_off, C)], pos_v, sem)
        cp_wrd = pltpu.async_copy(word_hbm.at[widx_v], acc_v, sem2)
        cp_typ = pltpu.async_copy(type_hbm.at[tidx_v], typ_v, sem3)
        cp_pos.wait()
        cp_wrd.wait()
        cp_typ.wait()

        def add_row(i, _):
            for g in range(GROUPS):
                sl = pl.ds(g * 16, 16)
                acc_v[i, sl] = acc_v[i, sl] + pos_v[i, sl] + typ_v[i, sl]
            return _

        lax.fori_loop(0, C, add_row, 0)
        pltpu.sync_copy(acc_v, out_hbm.at[pl.ds(base, C)])


@jax.jit
def _embed(ids, tt, word_table, type_table, pos_table):
    mesh = plsc.VectorSubcoreMesh(
        core_axis_name="c", subcore_axis_name="s", num_cores=NC, num_subcores=NS)
    k = pl.kernel(
        _embed_body,
        out_type=jax.ShapeDtypeStruct((TOK, HIDDEN), jnp.float32),
        mesh=mesh,
        scratch_types=[
            pltpu.VMEM((C,), jnp.int32),
            pltpu.VMEM((C,), jnp.int32),
            pltpu.VMEM((C, HIDDEN), jnp.float32),
            pltpu.VMEM((C, HIDDEN), jnp.float32),
            pltpu.VMEM((C, HIDDEN), jnp.float32),
            pltpu.SemaphoreType.DMA,
            pltpu.SemaphoreType.DMA,
            pltpu.SemaphoreType.DMA,
        ],
    )
    return k(ids, tt, word_table, type_table, pos_table)


def kernel(input_ids, token_type_ids, word_table, type_table, pos_table):
    ids = input_ids.reshape(-1)
    tt = token_type_ids.reshape(-1)
    out = _embed(ids, tt, word_table, type_table, pos_table)
    return out.reshape(BATCH, SEQ, HIDDEN)
